# Initial kernel scaffold; baseline (speedup 1.0000x reference)
import jax
import jax.numpy as jnp
from jax.experimental import pallas as pl


def kernel(features, labels, covariance, mean, amount):
    raise NotImplementedError("write your pallas kernel here")


# speedup vs baseline: 1.7555x; 1.7555x over previous
import jax
import jax.numpy as jnp
from jax import lax
from jax.experimental import pallas as pl
from jax.experimental.pallas import tpu as pltpu
from jax.experimental.pallas import tpu_sc as plsc

N = 128
A = 512
C = 1000
L = 16
NW = 32
W = A // NW

MOMENTUM = 0.8


def _body(feat_hbm, lab_hbm, cov_hbm, mean_hbm, amt_hbm,
          cov_out, mean_out, amt_out,
          lab_v, feat_v, cov_blk, mean_blk, amt_v, cnt_blk,
          acc_sum, acc_sq, res_cov, res_mean, amt_new_v):
    nc = 2
    wid = lax.axis_index("s") * nc + lax.axis_index("c")
    cb = wid * W

    pltpu.sync_copy(lab_hbm, lab_v.at[pl.ds(0, N)])
    pltpu.sync_copy(feat_hbm.at[:, pl.ds(cb, W)], feat_v)
    pltpu.sync_copy(amt_hbm, amt_v)
    pltpu.sync_copy(cov_hbm.at[:, pl.ds(cb, W)], cov_blk)
    pltpu.sync_copy(mean_hbm.at[:, pl.ds(cb, W)], mean_blk)

    zeros = jnp.zeros((L,), jnp.float32)
    ones = jnp.ones((L,), jnp.float32)

    def lab_at(n):
        return lab_v[pl.ds(n, L)][0]

    def zero_rows(n, _):
        l = lab_at(n)
        cnt_blk[l, :] = zeros
        acc_sum[l, :] = zeros
        acc_sq[l, :] = zeros
        return 0
    lax.fori_loop(0, N, zero_rows, 0)

    def accum(n, _):
        l = lab_at(n)
        cnt_blk[l, :] = cnt_blk[l, :] + ones
        acc_sum[l, :] = acc_sum[l, :] + feat_v[n, :]
        return 0
    lax.fori_loop(0, N, accum, 0)

    def accum_sq(n, _):
        l = lab_at(n)
        ave = acc_sum[l, :] / cnt_blk[l, :]
        d = feat_v[n, :] - ave
        acc_sq[l, :] = acc_sq[l, :] + d * d
        return 0
    lax.fori_loop(0, N, accum_sq, 0)

    def update(n, _):
        l = lab_at(n)
        cnt = cnt_blk[l, :]
        amt = plsc.load_gather(amt_v, [jnp.full((L,), l, jnp.int32)])
        w = cnt / (cnt + amt)
        w = jnp.maximum(w, 1.0 - MOMENTUM)
        ave = acc_sum[l, :] / cnt
        var = acc_sq[l, :] / cnt
        m = mean_blk[l, :]
        cv = cov_blk[l, :]
        dm = m - ave
        res_cov[n, :] = cv * (1.0 - w) + var * w + w * (1.0 - w) * dm * dm
        res_mean[n, :] = m * (1.0 - w) + ave * w
        return 0
    lax.fori_loop(0, N, update, 0)

    def scatter(n, _):
        l = lab_at(n)
        cov_blk[l, :] = res_cov[n, :]
        mean_blk[l, :] = res_mean[n, :]
        return 0
    lax.fori_loop(0, N, scatter, 0)

    pltpu.sync_copy(cov_blk, cov_out.at[:, pl.ds(cb, W)])
    pltpu.sync_copy(mean_blk, mean_out.at[:, pl.ds(cb, W)])

    @pl.when(wid == 0)
    def _():
        def copy_chunk(i, _):
            s = pl.ds(i * L, L)
            amt_new_v[s] = amt_v[s]
            return 0
        lax.fori_loop(0, C // L, copy_chunk, 0)
        s = pl.ds(C - L, L)
        amt_new_v[s] = amt_v[s]

        lanes = lax.iota(jnp.int32, L)
        def patch_chunk(i, _):
            lab16 = lab_v[pl.ds(i * L, L)]
            cnt16 = plsc.load_gather(cnt_blk, [lab16, lanes])
            amt16 = plsc.load_gather(amt_v, [lab16])
            plsc.store_scatter(amt_new_v, [lab16], amt16 + cnt16)
            return 0
        lax.fori_loop(0, N // L, patch_chunk, 0)
        pltpu.sync_copy(amt_new_v, amt_out)


_sc_call = pl.kernel(
    _body,
    out_type=(
        jax.ShapeDtypeStruct((C, A), jnp.float32),
        jax.ShapeDtypeStruct((C, A), jnp.float32),
        jax.ShapeDtypeStruct((C,), jnp.float32),
    ),
    mesh=plsc.VectorSubcoreMesh(core_axis_name="c", subcore_axis_name="s"),
    compiler_params=pltpu.CompilerParams(use_tc_tiling_on_sc=False,
                                          needs_layout_passes=False),
    scratch_types=[
        pltpu.VMEM((N + L,), jnp.int32),
        pltpu.VMEM((N, W), jnp.float32),
        pltpu.VMEM((C, W), jnp.float32),
        pltpu.VMEM((C, W), jnp.float32),
        pltpu.VMEM((C,), jnp.float32),
        pltpu.VMEM((C, W), jnp.float32),
        pltpu.VMEM((C, W), jnp.float32),
        pltpu.VMEM((C, W), jnp.float32),
        pltpu.VMEM((N, W), jnp.float32),
        pltpu.VMEM((N, W), jnp.float32),
        pltpu.VMEM((C,), jnp.float32),
    ],
)


@jax.jit
def kernel(features, labels, covariance, mean, amount):
    return _sc_call(features, labels, covariance, mean, amount)


# speedup vs baseline: 2.1572x; 1.2288x over previous
import jax
import jax.numpy as jnp
from jax import lax
from jax.experimental import pallas as pl
from jax.experimental.pallas import tpu as pltpu
from jax.experimental.pallas import tpu_sc as plsc

N = 128
A = 512
C = 1000
L = 16
NW = 32
W = A // NW

MOMENTUM = 0.8


def _body(feat_hbm, lab_hbm, cov_hbm, mean_hbm, amt_hbm,
          cov_out, mean_out, amt_out,
          lab_v, feat_v, cov_blk, mean_blk, amt_v, cnt_blk,
          acc_sum, acc_sq, amt_new_v,
          sem_lab, sem_feat, sem_amt, sem_cov, sem_mean):
    nc = 2
    wid = lax.axis_index("s") * nc + lax.axis_index("c")
    cb = wid * W

    c_lab = pltpu.async_copy(lab_hbm, lab_v.at[pl.ds(0, N)], sem_lab)
    c_feat = pltpu.async_copy(feat_hbm.at[:, pl.ds(cb, W)], feat_v, sem_feat)
    c_amt = pltpu.async_copy(amt_hbm, amt_v, sem_amt)
    c_cov = pltpu.async_copy(cov_hbm.at[:, pl.ds(cb, W)], cov_blk, sem_cov)
    c_mean = pltpu.async_copy(mean_hbm.at[:, pl.ds(cb, W)], mean_blk, sem_mean)

    zeros = jnp.zeros((L,), jnp.float32)
    ones = jnp.ones((L,), jnp.float32)

    def lab_at(n):
        return lab_v[pl.ds(n, L)][0]

    c_lab.wait()

    def zero_rows(n, _):
        l = lab_at(n)
        cnt_blk[l, :] = zeros
        acc_sum[l, :] = zeros
        acc_sq[l, :] = zeros
        return 0
    lax.fori_loop(0, N, zero_rows, 0, unroll=2)

    c_feat.wait()

    def accum(n, _):
        l = lab_at(n)
        f = feat_v[n, :]
        cnt_blk[l, :] = cnt_blk[l, :] + ones
        acc_sum[l, :] = acc_sum[l, :] + f
        acc_sq[l, :] = acc_sq[l, :] + f * f
        return 0
    lax.fori_loop(0, N, accum, 0, unroll=2)

    c_amt.wait()

    @pl.when(wid == 0)
    def _():
        def copy_chunk(i, _):
            s = pl.ds(i * L, L)
            amt_new_v[s] = amt_v[s]
            return 0
        lax.fori_loop(0, C // L, copy_chunk, 0, unroll=2)
        s = pl.ds(C - L, L)
        amt_new_v[s] = amt_v[s]

        lanes = lax.iota(jnp.int32, L)
        def patch_chunk(i, _):
            lab16 = lab_v[pl.ds(i * L, L)]
            cnt16 = plsc.load_gather(cnt_blk, [lab16, lanes])
            amt16 = plsc.load_gather(amt_v, [lab16])
            plsc.store_scatter(amt_new_v, [lab16], amt16 + cnt16)
            return 0
        lax.fori_loop(0, N // L, patch_chunk, 0)
        pltpu.async_copy(amt_new_v, amt_out, sem_amt)

    c_cov.wait()
    c_mean.wait()

    def update(n, _):
        l = lab_at(n)
        cnt = cnt_blk[l, :]

        @pl.when(cnt[0] > 0.0)
        def _():
            amt = plsc.load_gather(amt_v, [jnp.full((L,), l, jnp.int32)])
            w = cnt / (cnt + amt)
            w = jnp.maximum(w, 1.0 - MOMENTUM)
            rc = 1.0 / cnt
            ave = acc_sum[l, :] * rc
            var = acc_sq[l, :] * rc - ave * ave
            m = mean_blk[l, :]
            cv = cov_blk[l, :]
            dm = m - ave
            cov_blk[l, :] = cv * (1.0 - w) + var * w + w * (1.0 - w) * dm * dm
            mean_blk[l, :] = m * (1.0 - w) + ave * w
            cnt_blk[l, :] = -cnt
        return 0
    lax.fori_loop(0, N, update, 0)

    c_cov_o = pltpu.async_copy(cov_blk, cov_out.at[:, pl.ds(cb, W)], sem_cov)
    c_mean_o = pltpu.async_copy(mean_blk, mean_out.at[:, pl.ds(cb, W)], sem_mean)
    c_cov_o.wait()
    c_mean_o.wait()

    @pl.when(wid == 0)
    def _():
        pltpu.make_async_copy(amt_new_v, amt_out, sem_amt).wait()


_sc_call = pl.kernel(
    _body,
    out_type=(
        jax.ShapeDtypeStruct((C, A), jnp.float32),
        jax.ShapeDtypeStruct((C, A), jnp.float32),
        jax.ShapeDtypeStruct((C,), jnp.float32),
    ),
    mesh=plsc.VectorSubcoreMesh(core_axis_name="c", subcore_axis_name="s"),
    compiler_params=pltpu.CompilerParams(use_tc_tiling_on_sc=False,
                                         needs_layout_passes=False),
    scratch_types=[
        pltpu.VMEM((N + L,), jnp.int32),
        pltpu.VMEM((N, W), jnp.float32),
        pltpu.VMEM((C, W), jnp.float32),
        pltpu.VMEM((C, W), jnp.float32),
        pltpu.VMEM((C,), jnp.float32),
        pltpu.VMEM((C, W), jnp.float32),
        pltpu.VMEM((C, W), jnp.float32),
        pltpu.VMEM((C, W), jnp.float32),
        pltpu.VMEM((C,), jnp.float32),
        pltpu.SemaphoreType.DMA,
        pltpu.SemaphoreType.DMA,
        pltpu.SemaphoreType.DMA,
        pltpu.SemaphoreType.DMA,
        pltpu.SemaphoreType.DMA,
    ],
)


@jax.jit
def kernel(features, labels, covariance, mean, amount):
    return _sc_call(features, labels, covariance, mean, amount)


# speedup vs baseline: 2.3499x; 1.0893x over previous
import jax
import jax.numpy as jnp
from jax import lax
from jax.experimental import pallas as pl
from jax.experimental.pallas import tpu as pltpu
from jax.experimental.pallas import tpu_sc as plsc

N = 128
A = 512
C = 1000
L = 16
NW = 32
W = A // NW
CP = 1024

MOMENTUM = 0.8


def _body(feat_hbm, lab_hbm, cov_hbm, mean_hbm, amt_hbm,
          cov_out, mean_out, amt_out,
          lab_v, feat_v, cov_blk, mean_blk, amt_v, cnt_blk,
          acc_sum, acc_sq, amt_new_v, cov_new_c, mean_new_c,
          sem_lab, sem_feat, sem_amt, sem_cov, sem_mean):
    nc = 2
    wid = lax.axis_index("s") * nc + lax.axis_index("c")
    cb = wid * W

    c_lab = pltpu.async_copy(lab_hbm, lab_v, sem_lab)
    c_feat = pltpu.async_copy(feat_hbm.at[:, pl.ds(cb, W)], feat_v, sem_feat)
    c_amt = pltpu.async_copy(amt_hbm, amt_v.at[pl.ds(0, C)], sem_amt)
    c_cov = pltpu.async_copy(cov_hbm.at[:, pl.ds(cb, W)], cov_blk, sem_cov)
    c_mean = pltpu.async_copy(mean_hbm.at[:, pl.ds(cb, W)], mean_blk, sem_mean)

    zeros = jnp.zeros((L,), jnp.float32)
    ones = jnp.ones((L,), jnp.float32)

    c_lab.wait()

    @plsc.parallel_loop(0, N // L, unroll=2)
    def _(i):
        lab16 = lab_v[pl.ds(i * L, L)]
        for j in range(L):
            l = lab16[j]
            cnt_blk[l, :] = zeros
            acc_sum[l, :] = zeros
            acc_sq[l, :] = zeros

    c_feat.wait()

    def accum(i, _):
        lab16 = lab_v[pl.ds(i * L, L)]
        for j in range(L):
            l = lab16[j]
            f = feat_v[i * L + j, :]
            cnt_blk[l, :] = cnt_blk[l, :] + ones
            acc_sum[l, :] = acc_sum[l, :] + f
            acc_sq[l, :] = acc_sq[l, :] + f * f
        return 0
    lax.fori_loop(0, N // L, accum, 0)

    c_amt.wait()

    rb = wid * 2 * L
    a0 = amt_v[pl.ds(rb, L)]
    a1 = amt_v[pl.ds(rb + L, L)]
    amt_new_v[pl.ds(rb, L)] = a0
    amt_new_v[pl.ds(rb + L, L)] = a1

    lanes = lax.iota(jnp.int32, L)

    @plsc.parallel_loop(0, N // L)
    def _(i):
        lab16 = lab_v[pl.ds(i * L, L)]
        cnt16 = plsc.load_gather(cnt_blk, [lab16, lanes])
        amt16 = plsc.load_gather(amt_v, [lab16])
        plsc.store_scatter(amt_new_v, [lab16], amt16 + cnt16)

    @pl.when(wid < NW - 1)
    def _():
        pltpu.sync_copy(amt_new_v.at[pl.ds(rb, 2 * L)],
                        amt_out.at[pl.ds(rb, 2 * L)])

    @pl.when(wid == NW - 1)
    def _():
        pltpu.sync_copy(amt_new_v.at[pl.ds(C - 8, 8)],
                        amt_out.at[pl.ds(C - 8, 8)])

    c_cov.wait()
    c_mean.wait()

    @plsc.parallel_loop(0, N // L, unroll=2)
    def _(i):
        lab16 = lab_v[pl.ds(i * L, L)]
        for j in range(L):
            l = lab16[j]
            n = i * L + j
            cnt = cnt_blk[l, :]
            amt = amt_v[pl.ds(l, L)][0]
            w = jnp.maximum(cnt / (cnt + amt), 1.0 - MOMENTUM)
            rc = 1.0 / cnt
            ave = acc_sum[l, :] * rc
            var = acc_sq[l, :] * rc - ave * ave
            m = mean_blk[l, :]
            dm = m - ave
            omw = 1.0 - w
            cov_new_c[n, :] = (cov_blk[l, :] * omw + var * w
                               + w * omw * dm * dm)
            mean_new_c[n, :] = m * omw + ave * w

    @plsc.parallel_loop(0, N // L, unroll=2)
    def _(i):
        lab16 = lab_v[pl.ds(i * L, L)]
        for j in range(L):
            l = lab16[j]
            n = i * L + j
            cov_blk[l, :] = cov_new_c[n, :]
            mean_blk[l, :] = mean_new_c[n, :]

    c_cov_o = pltpu.async_copy(cov_blk, cov_out.at[:, pl.ds(cb, W)], sem_cov)
    c_mean_o = pltpu.async_copy(mean_blk, mean_out.at[:, pl.ds(cb, W)],
                                sem_mean)
    c_cov_o.wait()
    c_mean_o.wait()


_sc_call = pl.kernel(
    _body,
    out_type=(
        jax.ShapeDtypeStruct((C, A), jnp.float32),
        jax.ShapeDtypeStruct((C, A), jnp.float32),
        jax.ShapeDtypeStruct((C,), jnp.float32),
    ),
    mesh=plsc.VectorSubcoreMesh(core_axis_name="c", subcore_axis_name="s"),
    compiler_params=pltpu.CompilerParams(use_tc_tiling_on_sc=False,
                                         needs_layout_passes=False),
    scratch_types=[
        pltpu.VMEM((N,), jnp.int32),
        pltpu.VMEM((N, W), jnp.float32),
        pltpu.VMEM((C, W), jnp.float32),
        pltpu.VMEM((C, W), jnp.float32),
        pltpu.VMEM((CP,), jnp.float32),
        pltpu.VMEM((C, W), jnp.float32),
        pltpu.VMEM((C, W), jnp.float32),
        pltpu.VMEM((C, W), jnp.float32),
        pltpu.VMEM((CP,), jnp.float32),
        pltpu.VMEM((N, W), jnp.float32),
        pltpu.VMEM((N, W), jnp.float32),
        pltpu.SemaphoreType.DMA,
        pltpu.SemaphoreType.DMA,
        pltpu.SemaphoreType.DMA,
        pltpu.SemaphoreType.DMA,
        pltpu.SemaphoreType.DMA,
    ],
)


@jax.jit
def kernel(features, labels, covariance, mean, amount):
    return _sc_call(features, labels, covariance, mean, amount)
